# baseline (speedup 1.0000x reference)
import jax
import jax.numpy as jnp
from jax import lax
from jax.experimental import pallas as pl
from jax.experimental.pallas import tpu as pltpu
from jax.experimental.pallas import tpu_sc as plsc

NUM_EMBEDDINGS = 1000000
D = 64
B = 16384
K = 50

NC = 2
NS = 16
NW = NC * NS
LANES = 16
DV = D // LANES

BPW = B // NW
PAIR = 16
HALF = 8
NP = BPW // PAIR
CB = 128
PPCB = CB // PAIR
HK = HALF * K
G = 80
NG = HK // G
KS = 56


def _bag_body(idxT, scT, w2, out_hbm,
              idx_v, sc_v, list_v, sc0, sc1, of0, of1,
              rows_a, rows_b, out_v, sem_a, sem_b):
    wid = lax.axis_index("s") * NC + lax.axis_index("c")
    base = wid * BPW
    scs, ofs = (sc0, sc1), (of0, of1)
    ia_list = lax.iota(jnp.int32, LANES) * K
    ia_56 = lax.iota(jnp.int32, LANES) * KS

    def fetch_block(cb):
        col0 = pl.multiple_of(base + cb * CB, CB)
        pltpu.sync_copy(idxT.at[:, pl.ds(col0, CB)], idx_v)
        pltpu.sync_copy(scT.at[:, pl.ds(col0, CB)], sc_v)

    def build(p):
        q = lax.rem(p, 2)
        lane0 = pl.multiple_of(lax.rem(p, PPCB) * PAIR, PAIR)
        for k in range(K):
            iv = idx_v[k, pl.ds(lane0, LANES)]
            sv = sc_v[k, pl.ds(lane0, LANES)]
            plsc.store_scatter(list_v, [ia_list + k], iv >> 1)

            @pl.when(q == 0)
            def _():
                plsc.store_scatter(of0, [ia_56 + k], (iv & 1) << 6)
                plsc.store_scatter(sc0, [ia_56 + k], sv)

            @pl.when(q == 1)
            def _():
                plsc.store_scatter(of1, [ia_56 + k], (iv & 1) << 6)
                plsc.store_scatter(sc1, [ia_56 + k], sv)

    def fire(h, rows, sem):
        for i in range(NG):
            pltpu.async_copy(
                w2.at[list_v.at[pl.ds(h * HK + i * G, G)]],
                rows.at[pl.ds(i * G, G)], sem)

    def drain(h, rows, sem):
        for i in range(NG):
            pltpu.make_async_copy(
                w2.at[list_v.at[pl.ds(h * HK + i * G, G)]],
                rows.at[pl.ds(i * G, G)], sem).wait()

    def compute(p, h, rows, sc56, of56):
        def bag(c, carry):
            b56 = pl.multiple_of((h * HALF + c) * KS, 8)
            svs = [sc56[pl.ds(b56 + g * LANES, LANES)] for g in range(4)]
            ovs = [of56[pl.ds(b56 + g * LANES, LANES)] for g in range(4)]
            accs = [jnp.zeros((LANES,), jnp.float32) for _ in range(DV)]
            for k in range(K):
                s = svs[k // LANES][k % LANES]
                off = pl.multiple_of(ovs[k // LANES][k % LANES], 64)
                r = c * K + k
                for d in range(DV):
                    accs[d] = accs[d] + rows[r, pl.ds(off + d * LANES, LANES)] * s
            for d in range(DV):
                out_v[c, pl.ds(d * LANES, LANES)] = accs[d]
            return carry

        lax.fori_loop(0, HALF, bag, 0)
        bag0 = base + p * PAIR + h * HALF
        pltpu.sync_copy(out_v, out_hbm.at[pl.ds(pl.multiple_of(bag0, 8), HALF), :])

    fetch_block(0)
    build(0)
    fire(0, rows_a, sem_a)

    def step(it, carry):
        for q in range(2):
            p = 2 * it + q
            drain(0, rows_a, sem_a)
            fire(1, rows_b, sem_b)
            compute(p, 0, rows_a, scs[q], ofs[q])
            drain(1, rows_b, sem_b)

            @pl.when(p + 1 < NP)
            def _():
                @pl.when(lax.rem(p + 1, PPCB) == 0)
                def _():
                    fetch_block((p + 1) // PPCB)

                build(p + 1)
                fire(0, rows_a, sem_a)

            compute(p, 1, rows_b, scs[q], ofs[q])
        return carry

    lax.fori_loop(0, NP // 2, step, 0)


@jax.jit
def _run(idxT, scT, w2):
    mesh = plsc.VectorSubcoreMesh(core_axis_name="c", subcore_axis_name="s")
    return pl.kernel(
        _bag_body,
        out_type=jax.ShapeDtypeStruct((B, D), jnp.float32),
        mesh=mesh,
        compiler_params=pltpu.CompilerParams(needs_layout_passes=False),
        scratch_types=[
            pltpu.VMEM((K, CB), jnp.int32),
            pltpu.VMEM((K, CB), jnp.float32),
            pltpu.VMEM((PAIR * K,), jnp.int32),
            pltpu.VMEM((PAIR * KS,), jnp.float32),
            pltpu.VMEM((PAIR * KS,), jnp.float32),
            pltpu.VMEM((PAIR * KS,), jnp.int32),
            pltpu.VMEM((PAIR * KS,), jnp.int32),
            pltpu.VMEM((HK, 2 * D), jnp.float32),
            pltpu.VMEM((HK, 2 * D), jnp.float32),
            pltpu.VMEM((HALF, D), jnp.float32),
            pltpu.SemaphoreType.DMA,
            pltpu.SemaphoreType.DMA,
        ],
    )(idxT, scT, w2)


def kernel(indices, scores, weight):
    idxT = indices.astype(jnp.int32).T
    scT = scores.T
    h = NUM_EMBEDDINGS // 2
    w2 = jnp.concatenate(
        [weight[:h].reshape(h // 2, 2 * D), weight[h:].reshape(h // 2, 2 * D)],
        axis=0)
    return _run(idxT, scT, w2)


# speedup vs baseline: 1.6770x; 1.6770x over previous
import jax
import jax.numpy as jnp
from jax import lax
from jax.experimental import pallas as pl
from jax.experimental.pallas import tpu as pltpu
from jax.experimental.pallas import tpu_sc as plsc

NUM_EMBEDDINGS = 1000000
D = 64
B = 16384
K = 50

NC = 2
NS = 16
NW = NC * NS
LANES = 16
DV = D // LANES

BPW = B // NW
PAIR = 16
HALF = 8
NP = BPW // PAIR
CB = 128
PPCB = CB // PAIR
HK = HALF * K
G = 80
NG = HK // G
KS = 56


def _bag_body(idxT, scT, w2, out_hbm,
              idx_v, sc_v, list_v, sc0, sc1,
              rows_a, rows_b, out_v, sem_a, sem_b):
    wid = lax.axis_index("s") * NC + lax.axis_index("c")
    base = wid * BPW
    scs = (sc0, sc1)
    ia_list = lax.iota(jnp.int32, LANES) * K
    ia_56 = lax.iota(jnp.int32, LANES) * KS

    def fetch_block(cb):
        col0 = pl.multiple_of(base + cb * CB, CB)
        pltpu.sync_copy(idxT.at[:, pl.ds(col0, CB)], idx_v)
        pltpu.sync_copy(scT.at[:, pl.ds(col0, CB)], sc_v)

    def build(p):
        q = lax.rem(p, 2)
        lane0 = pl.multiple_of(lax.rem(p, PPCB) * PAIR, PAIR)
        for k in range(K):
            iv = idx_v[k, pl.ds(lane0, LANES)]
            sv = sc_v[k, pl.ds(lane0, LANES)]
            plsc.store_scatter(list_v, [ia_list + k], iv)

            @pl.when(q == 0)
            def _():
                plsc.store_scatter(sc0, [ia_56 + k], sv)

            @pl.when(q == 1)
            def _():
                plsc.store_scatter(sc1, [ia_56 + k], sv)

    def fire(h, rows, sem):
        for i in range(NG):
            pltpu.async_copy(
                w2.at[list_v.at[pl.ds(h * HK + i * G, G)]],
                rows.at[pl.ds(i * G, G)], sem)

    def drain(h, rows, sem):
        for i in range(NG):
            pltpu.make_async_copy(
                w2.at[list_v.at[pl.ds(h * HK + i * G, G)]],
                rows.at[pl.ds(i * G, G)], sem).wait()

    def compute(p, h, rows, sc56):
        def bag(c, carry):
            b56 = pl.multiple_of((h * HALF + c) * KS, 8)
            svs = [sc56[pl.ds(b56 + g * LANES, LANES)] for g in range(4)]
            accs = [jnp.zeros((LANES,), jnp.float32) for _ in range(DV)]
            for k in range(K):
                s = svs[k // LANES][k % LANES]
                r = c * K + k
                for d in range(DV):
                    accs[d] = accs[d] + rows[r, pl.ds(d * LANES, LANES)] * s
            for d in range(DV):
                out_v[c, pl.ds(d * LANES, LANES)] = accs[d]
            return carry

        lax.fori_loop(0, HALF, bag, 0)
        bag0 = base + p * PAIR + h * HALF
        pltpu.sync_copy(out_v, out_hbm.at[pl.ds(pl.multiple_of(bag0, 8), HALF), :])

    fetch_block(0)
    build(0)
    fire(0, rows_a, sem_a)

    def step(it, carry):
        for q in range(2):
            p = 2 * it + q
            drain(0, rows_a, sem_a)
            fire(1, rows_b, sem_b)
            compute(p, 0, rows_a, scs[q])
            drain(1, rows_b, sem_b)

            @pl.when(p + 1 < NP)
            def _():
                @pl.when(lax.rem(p + 1, PPCB) == 0)
                def _():
                    fetch_block((p + 1) // PPCB)

                build(p + 1)
                fire(0, rows_a, sem_a)

            compute(p, 1, rows_b, scs[q])
        return carry

    lax.fori_loop(0, NP // 2, step, 0)


@jax.jit
def _run(idxT, scT, w2):
    mesh = plsc.VectorSubcoreMesh(core_axis_name="c", subcore_axis_name="s")
    return pl.kernel(
        _bag_body,
        out_type=jax.ShapeDtypeStruct((B, D), jnp.float32),
        mesh=mesh,
        compiler_params=pltpu.CompilerParams(
            needs_layout_passes=False, use_tc_tiling_on_sc=False),
        scratch_types=[
            pltpu.VMEM((K, CB), jnp.int32),
            pltpu.VMEM((K, CB), jnp.float32),
            pltpu.VMEM((PAIR * K,), jnp.int32),
            pltpu.VMEM((PAIR * KS,), jnp.float32),
            pltpu.VMEM((PAIR * KS,), jnp.float32),
            pltpu.VMEM((HK, D), jnp.float32),
            pltpu.VMEM((HK, D), jnp.float32),
            pltpu.VMEM((HALF, D), jnp.float32),
            pltpu.SemaphoreType.DMA,
            pltpu.SemaphoreType.DMA,
        ],
    )(idxT, scT, w2)


def kernel(indices, scores, weight):
    idxT = indices.astype(jnp.int32).T
    scT = scores.T
    return _run(idxT, scT, weight)
